# baseline (speedup 1.0000x reference)
import jax
import jax.numpy as jnp
from jax.experimental import pallas as pl

B, N, D = 512, 64, 64
K = 16
HID = 64
BASIS = 16
C = D + BASIS
DEPTH = 2
BT = 1


def _silu(u):
    return u * jax.nn.sigmoid(u)


def _body(z_ref, x_ref, We1_ref, be1_ref, We2_ref, be2_ref, WxT_ref, bx_ref,
          Wh1_ref, bh1_ref, Wh2_ref, bh2_ref, cen_ref, gam_ref,
          Wq_ref, bq_ref, Wk_ref, bk_ref, Wv_ref, bv_ref, Wo_ref, bo_ref,
          Wa_ref, ba_ref, Wb_ref, bb_ref, out_ref):
    h = z_ref[...]
    x = x_ref[...]

    for l in range(DEPTH):
        We1 = We1_ref[l]
        Wa1 = We1[:D]
        Wb1 = We1[D:2 * D]
        w1 = We1[2 * D:2 * D + 1]
        be1 = be1_ref[l]
        We2 = We2_ref[l]
        be2 = be2_ref[l]
        WxT = WxT_ref[l]
        bx = bx_ref[l]
        Wh1 = Wh1_ref[l]
        bh1 = bh1_ref[l]
        Wh2 = Wh2_ref[l]
        bh2 = bh2_ref[l]

        diff = x[:, :, None, :] - x[:, None, :, :]
        dist2 = jnp.sum(diff * diff, axis=-1)
        ii = jax.lax.broadcasted_iota(jnp.int32, (N, N), 0)
        jj = jax.lax.broadcasted_iota(jnp.int32, (N, N), 1)
        d = dist2 + jnp.where(ii == jj, 1e9, 0.0)[None]

        dj = d[:, :, :, None]
        dk = d[:, :, None, :]
        kio = jax.lax.broadcasted_iota(jnp.int32, (1, 1, N, N), 3)
        jio = jax.lax.broadcasted_iota(jnp.int32, (1, 1, N, N), 2)
        sel = (dk < dj) | ((dk == dj) & (kio < jio))
        rank = jnp.sum(jnp.where(sel, 1.0, 0.0), axis=3)
        mask = jnp.where(rank < K, 1.0, 0.0)

        h2 = h.reshape(BT * N, D)
        u = (h2 @ Wa1).reshape(BT, N, HID)
        v = (h2 @ Wb1).reshape(BT, N, HID)
        pre1 = (u[:, :, None, :] + v[:, None, :, :]
                + dist2[..., None] * w1.reshape(1, 1, 1, HID)
                + be1.reshape(1, 1, 1, HID))
        m1 = _silu(pre1)
        m2 = _silu((m1.reshape(BT * N * N, HID) @ We2)
                   .reshape(BT, N, N, HID) + be2.reshape(1, 1, 1, HID))

        agg = jnp.sum(m2 * mask[..., None], axis=2)
        coef = (jnp.sum(m2 * WxT.reshape(1, 1, 1, HID), axis=-1, keepdims=True)
                + bx[0, 0])
        rel_n = diff / (jnp.sqrt(dist2)[..., None] + 1.0)
        x = x + jnp.sum(rel_n * coef * mask[..., None], axis=2) * (1.0 / K)

        W1a = Wh1[:D]
        W1b = Wh1[D:]
        t1 = _silu(h2 @ W1a + agg.reshape(BT * N, HID) @ W1b + bh1)
        h = h + (t1 @ Wh2 + bh2).reshape(BT, N, D)

    cent = jnp.mean(x, axis=1, keepdims=True)
    cd = x - cent
    dist = jnp.sqrt(jnp.sum(cd * cd, axis=-1))
    gam = gam_ref[...].reshape(1, 1, BASIS)
    cen = cen_ref[...].reshape(1, 1, BASIS)
    r = jnp.exp(-gam * (dist[:, :, None] - cen) ** 2)

    h2 = h.reshape(BT * N, D)
    r2 = r.reshape(BT * N, BASIS)
    Wq = Wq_ref[...]
    Wk = Wk_ref[...]
    Wv = Wv_ref[...]
    q = (r2 @ Wq[:BASIS] + h2 @ Wq[BASIS:] + bq_ref[...]).reshape(BT, N, C)
    k_ = (r2 @ Wk[:BASIS] + h2 @ Wk[BASIS:] + bk_ref[...]).reshape(BT, N, C)
    v_ = (r2 @ Wv[:BASIS] + h2 @ Wv[BASIS:] + bv_ref[...]).reshape(BT, N, C)

    scores = jax.lax.dot_general(
        q, k_, (((2,), (2,)), ((0,), (0,)))) * (1.0 / jnp.sqrt(jnp.float32(C)))
    smax = jnp.max(scores, axis=-1, keepdims=True)
    e = jnp.exp(scores - smax)
    a = e / jnp.sum(e, axis=-1, keepdims=True)
    att = jax.lax.dot_general(a, v_, (((2,), (1,)), ((0,), (0,))))
    att2 = att.reshape(BT * N, C) @ Wo_ref[...] + bo_ref[...]
    s = att2 @ Wa_ref[...] + ba_ref[...]
    preds = jnp.max(s.reshape(BT, N), axis=1, keepdims=True)
    out_ref[...] = (preds * Wb_ref[0, 0] + bb_ref[0, 0])[None]


def kernel(z, x, We1, be1, We2, be2, Wx, bx, Wh1, bh1, Wh2, bh2,
           rbf_centers, rbf_gamma, Wq, bq, Wk, bk, Wv, bv, Wo, bo,
           Wa, ba, Wb, bb):
    be1 = be1.reshape(DEPTH, 1, HID)
    be2 = be2.reshape(DEPTH, 1, HID)
    WxT = jnp.transpose(Wx, (0, 2, 1))
    bx = bx.reshape(DEPTH, 1, 1)
    bh1 = bh1.reshape(DEPTH, 1, HID)
    bh2 = bh2.reshape(DEPTH, 1, D)
    cen = rbf_centers.reshape(1, BASIS)
    gam = rbf_gamma.reshape(1, BASIS)
    bq = bq.reshape(1, C)
    bk = bk.reshape(1, C)
    bv = bv.reshape(1, C)
    bo = bo.reshape(1, C)
    ba = ba.reshape(1, 1)
    bb = bb.reshape(1, 1)

    args = (z, x, We1, be1, We2, be2, WxT, bx, Wh1, bh1, Wh2, bh2,
            cen, gam, Wq, bq, Wk, bk, Wv, bv, Wo, bo, Wa, ba, Wb, bb)

    def spec(arr, blocked=False):
        if blocked:
            blk = (BT,) + arr.shape[1:]
            return pl.BlockSpec(blk, lambda i: (i,) + (0,) * (arr.ndim - 1))
        return pl.BlockSpec(arr.shape, lambda i: (0,) * arr.ndim)

    in_specs = [spec(z, True), spec(x, True)] + [spec(a) for a in args[2:]]

    out = pl.pallas_call(
        _body,
        grid=(B // BT,),
        in_specs=in_specs,
        out_specs=pl.BlockSpec((1, BT, 1), lambda i: (i, 0, 0)),
        out_shape=jax.ShapeDtypeStruct((B // BT, BT, 1), jnp.float32),
    )(*args)
    return out.reshape(B, 1)


# speedup vs baseline: 4.1526x; 4.1526x over previous
import jax
import jax.numpy as jnp
from jax.experimental import pallas as pl

B, N, D = 512, 64, 64
K = 16
HID = 64
BASIS = 16
C = D + BASIS
DEPTH = 2
NK = N * K


def _silu(u):
    return u * jax.nn.sigmoid(u)


_HI = jax.lax.Precision.HIGHEST


def _mm(a, b):
    return jax.lax.dot_general(a, b, (((1,), (0,)), ((), ())), precision=_HI)


def _body(z_ref, x_ref, We1_ref, be1_ref, We2_ref, be2_ref, Wx_ref, bx_ref,
          Wh1_ref, bh1_ref, Wh2_ref, bh2_ref, cen_ref, gam_ref,
          Wq_ref, bq_ref, Wk_ref, bk_ref, Wv_ref, bv_ref, Wo_ref, bo_ref,
          Wa_ref, ba_ref, Wb_ref, bb_ref, out_ref):
    h = z_ref[0]
    x = x_ref[0]

    ii = jax.lax.broadcasted_iota(jnp.int32, (N, N), 0)
    jj = jax.lax.broadcasted_iota(jnp.int32, (N, N), 1)
    diag = jnp.where(ii == jj, 1e9, 0.0)
    eyeN = jnp.where(ii == jj, 1.0, 0.0)
    si = jax.lax.broadcasted_iota(jnp.int32, (N, NK), 0)
    sj = jax.lax.broadcasted_iota(jnp.int32, (N, NK), 1)
    S_seg = jnp.where(si == sj // K, 1.0, 0.0)
    kio = jax.lax.broadcasted_iota(jnp.int32, (N, K, 1), 1).reshape(NK, 1)

    for l in range(DEPTH):
        We1 = We1_ref[l]
        Wa1 = We1[:D]
        Wb1 = We1[D:2 * D]
        w1 = We1[2 * D:2 * D + 1]
        be1 = be1_ref[l]
        We2 = We2_ref[l]
        be2 = be2_ref[l]
        Wx_l = Wx_ref[l]
        bx = bx_ref[l]
        Wh1 = Wh1_ref[l]
        bh1 = bh1_ref[l]
        Wh2 = Wh2_ref[l]
        bh2 = bh2_ref[l]

        xT = jax.lax.dot_general(x, eyeN, (((0,), (0,)), ((), ())),
                                 precision=_HI)
        d0 = x[:, 0:1] - xT[0:1, :]
        d1 = x[:, 1:2] - xT[1:2, :]
        d2c = x[:, 2:3] - xT[2:3, :]
        dist2 = (d0 * d0 + d1 * d1) + d2c * d2c
        d = dist2 + diag

        T = jnp.where(d[:, None, :] < d[:, :, None], 1.0, 0.0)
        rank = jnp.sum(T, axis=2)
        rank_b = jnp.broadcast_to(rank[:, None, :], (N, K, N)).reshape(NK, N)
        P2 = jnp.where(rank_b == kio.astype(jnp.float32), 1.0, 0.0)

        xj = _mm(P2, x)
        hj_pre = _mm(P2, _mm(h, Wb1))
        u = _mm(h, Wa1)
        u_b = jnp.broadcast_to(u[:, None, :], (N, K, HID)).reshape(NK, HID)
        xi_b = jnp.broadcast_to(x[:, None, :], (N, K, 3)).reshape(NK, 3)
        rel = xi_b - xj
        d2s = jnp.sum(rel * rel, axis=1, keepdims=True)

        m1 = _silu(u_b + hj_pre + d2s * w1 + be1)
        m2 = _silu(_mm(m1, We2) + be2)

        agg = _mm(S_seg, m2)
        coef = _mm(m2, Wx_l) + bx
        wgt = coef / (jnp.sqrt(d2s) + 1.0) * (1.0 / K)
        x = x + _mm(S_seg, rel * wgt)

        t1 = _silu(_mm(h, Wh1[:D]) + _mm(agg, Wh1[D:]) + bh1)
        h = h + _mm(t1, Wh2) + bh2

    cent = jnp.mean(x, axis=0, keepdims=True)
    cd = x - cent
    dist = jnp.sqrt(jnp.sum(cd * cd, axis=1, keepdims=True))
    r = jnp.exp(-gam_ref[...] * (dist - cen_ref[...]) ** 2)

    Wq = Wq_ref[...]
    Wk = Wk_ref[...]
    Wv = Wv_ref[...]
    q = _mm(r, Wq[:BASIS]) + _mm(h, Wq[BASIS:]) + bq_ref[...]
    k_ = _mm(r, Wk[:BASIS]) + _mm(h, Wk[BASIS:]) + bk_ref[...]
    v_ = _mm(r, Wv[:BASIS]) + _mm(h, Wv[BASIS:]) + bv_ref[...]

    scores = jax.lax.dot_general(
        q, k_, (((1,), (1,)), ((), ())),
        precision=_HI) * (1.0 / jnp.sqrt(jnp.float32(C)))
    smax = jnp.max(scores, axis=-1, keepdims=True)
    e = jnp.exp(scores - smax)
    a = e / jnp.sum(e, axis=-1, keepdims=True)
    att = _mm(a, v_)
    att2 = _mm(att, Wo_ref[...]) + bo_ref[...]
    s = _mm(att2, Wa_ref[...]) + ba_ref[...]
    pred = jnp.max(s, axis=0, keepdims=True)
    out_ref[...] = (pred * Wb_ref[0, 0] + bb_ref[0, 0])[None]


def kernel(z, x, We1, be1, We2, be2, Wx, bx, Wh1, bh1, Wh2, bh2,
           rbf_centers, rbf_gamma, Wq, bq, Wk, bk, Wv, bv, Wo, bo,
           Wa, ba, Wb, bb):
    be1 = be1.reshape(DEPTH, 1, HID)
    be2 = be2.reshape(DEPTH, 1, HID)
    bx = bx.reshape(DEPTH, 1, 1)
    bh1 = bh1.reshape(DEPTH, 1, HID)
    bh2 = bh2.reshape(DEPTH, 1, D)
    cen = rbf_centers.reshape(1, BASIS)
    gam = rbf_gamma.reshape(1, BASIS)
    bq = bq.reshape(1, C)
    bk = bk.reshape(1, C)
    bv = bv.reshape(1, C)
    bo = bo.reshape(1, C)
    ba = ba.reshape(1, 1)
    bb = bb.reshape(1, 1)

    args = (z, x, We1, be1, We2, be2, Wx, bx, Wh1, bh1, Wh2, bh2,
            cen, gam, Wq, bq, Wk, bk, Wv, bv, Wo, bo, Wa, ba, Wb, bb)

    def spec(arr, blocked=False):
        if blocked:
            blk = (1,) + arr.shape[1:]
            return pl.BlockSpec(blk, lambda i: (i,) + (0,) * (arr.ndim - 1))
        return pl.BlockSpec(arr.shape, lambda i: (0,) * arr.ndim)

    in_specs = [spec(z, True), spec(x, True)] + [spec(a) for a in args[2:]]

    out = pl.pallas_call(
        _body,
        grid=(B,),
        in_specs=in_specs,
        out_specs=pl.BlockSpec((1, 1, 1), lambda i: (i, 0, 0)),
        out_shape=jax.ShapeDtypeStruct((B, 1, 1), jnp.float32),
    )(*args)
    return out.reshape(B, 1)


# speedup vs baseline: 7.2712x; 1.7510x over previous
import jax
import jax.numpy as jnp
from jax.experimental import pallas as pl

B, N, D = 512, 64, 64
K = 16
HID = 64
BASIS = 16
C = D + BASIS
DEPTH = 2
NK = N * K


def _silu(u):
    return u * jax.nn.sigmoid(u)


_HI = jax.lax.Precision.HIGHEST


def _mm(a, b):
    return jax.lax.dot_general(a, b, (((1,), (0,)), ((), ())), precision=_HI)


def _body(z_ref, x_ref, We1_ref, be1_ref, We2_ref, be2_ref, Wx_ref, bx_ref,
          Wh1_ref, bh1_ref, Wh2_ref, bh2_ref, cen_ref, gam_ref,
          Wq_ref, bq_ref, Wk_ref, bk_ref, Wv_ref, bv_ref, Wo_ref, bo_ref,
          Wa_ref, ba_ref, Wb_ref, bb_ref,
          diag_ref, eyeN_ref, S_seg_ref, R16_ref, kio_ref, out_ref):
    h = z_ref[0]
    x = x_ref[0]

    diag = diag_ref[...]
    eyeN = eyeN_ref[...]
    S_seg = S_seg_ref[...]
    R16 = R16_ref[...]
    kio = kio_ref[...]

    for l in range(DEPTH):
        We1 = We1_ref[l]
        Wa1 = We1[:D]
        Wb1 = We1[D:2 * D]
        w1 = We1[2 * D:2 * D + 1]
        be1 = be1_ref[l]
        We2 = We2_ref[l]
        be2 = be2_ref[l]
        Wx_l = Wx_ref[l]
        bx = bx_ref[l]
        Wh1 = Wh1_ref[l]
        bh1 = bh1_ref[l]
        Wh2 = Wh2_ref[l]
        bh2 = bh2_ref[l]

        xT = jax.lax.dot_general(x, eyeN, (((0,), (0,)), ((), ())),
                                 precision=_HI)
        d0 = x[:, 0:1] - xT[0:1, :]
        d1 = x[:, 1:2] - xT[1:2, :]
        d2c = x[:, 2:3] - xT[2:3, :]
        dist2 = (d0 * d0 + d1 * d1) + d2c * d2c
        d = dist2 + diag

        T = jnp.where(d[:, None, :] < d[:, :, None], 1.0, 0.0)
        rank = jnp.sum(T, axis=2)
        rank_b = jax.lax.dot_general(
            R16, rank, (((1,), (0,)), ((), ())))
        P2 = jnp.where(rank_b == kio, 1.0, 0.0)

        xj = _mm(P2, x)
        hj_pre = _mm(P2, _mm(h, Wb1))
        u = _mm(h, Wa1)
        u_b = _mm(R16, u)
        xi_b = _mm(R16, x)
        rel = xi_b - xj
        d2s = jnp.sum(rel * rel, axis=1, keepdims=True)

        m1 = _silu(u_b + hj_pre + d2s * w1 + be1)
        m2 = _silu(_mm(m1, We2) + be2)

        agg = _mm(S_seg, m2)
        coef = _mm(m2, Wx_l) + bx
        wgt = coef / (jnp.sqrt(d2s) + 1.0) * (1.0 / K)
        x = x + _mm(S_seg, rel * wgt)

        t1 = _silu(_mm(h, Wh1[:D]) + _mm(agg, Wh1[D:]) + bh1)
        h = h + _mm(t1, Wh2) + bh2

    cent = jnp.mean(x, axis=0, keepdims=True)
    cd = x - cent
    dist = jnp.sqrt(jnp.sum(cd * cd, axis=1, keepdims=True))
    r = jnp.exp(-gam_ref[...] * (dist - cen_ref[...]) ** 2)

    Wq = Wq_ref[...]
    Wk = Wk_ref[...]
    Wv = Wv_ref[...]
    q = _mm(r, Wq[:BASIS]) + _mm(h, Wq[BASIS:]) + bq_ref[...]
    k_ = _mm(r, Wk[:BASIS]) + _mm(h, Wk[BASIS:]) + bk_ref[...]
    v_ = _mm(r, Wv[:BASIS]) + _mm(h, Wv[BASIS:]) + bv_ref[...]

    scores = jax.lax.dot_general(
        q, k_, (((1,), (1,)), ((), ())),
        precision=_HI) * (1.0 / jnp.sqrt(jnp.float32(C)))
    smax = jnp.max(scores, axis=-1, keepdims=True)
    e = jnp.exp(scores - smax)
    a = e / jnp.sum(e, axis=-1, keepdims=True)
    att = _mm(a, v_)
    att2 = _mm(att, Wo_ref[...]) + bo_ref[...]
    s = _mm(att2, Wa_ref[...]) + ba_ref[...]
    pred = jnp.max(s, axis=0, keepdims=True)
    out_ref[...] = (pred * Wb_ref[0, 0] + bb_ref[0, 0])[None]


def kernel(z, x, We1, be1, We2, be2, Wx, bx, Wh1, bh1, Wh2, bh2,
           rbf_centers, rbf_gamma, Wq, bq, Wk, bk, Wv, bv, Wo, bo,
           Wa, ba, Wb, bb):
    be1 = be1.reshape(DEPTH, 1, HID)
    be2 = be2.reshape(DEPTH, 1, HID)
    bx = bx.reshape(DEPTH, 1, 1)
    bh1 = bh1.reshape(DEPTH, 1, HID)
    bh2 = bh2.reshape(DEPTH, 1, D)
    cen = rbf_centers.reshape(1, BASIS)
    gam = rbf_gamma.reshape(1, BASIS)
    bq = bq.reshape(1, C)
    bk = bk.reshape(1, C)
    bv = bv.reshape(1, C)
    bo = bo.reshape(1, C)
    ba = ba.reshape(1, 1)
    bb = bb.reshape(1, 1)

    ii = jnp.arange(N, dtype=jnp.int32)
    diag = jnp.where(ii[:, None] == ii[None, :], 1e9, 0.0).astype(jnp.float32)
    eyeN = jnp.eye(N, dtype=jnp.float32)
    sj = jnp.arange(NK, dtype=jnp.int32)
    S_seg = (ii[:, None] == sj[None, :] // K).astype(jnp.float32)
    R16 = S_seg.T
    kio = (sj % K).astype(jnp.float32).reshape(NK, 1)

    args = (z, x, We1, be1, We2, be2, Wx, bx, Wh1, bh1, Wh2, bh2,
            cen, gam, Wq, bq, Wk, bk, Wv, bv, Wo, bo, Wa, ba, Wb, bb,
            diag, eyeN, S_seg, R16, kio)

    def spec(arr, blocked=False):
        if blocked:
            blk = (1,) + arr.shape[1:]
            return pl.BlockSpec(blk, lambda i: (i,) + (0,) * (arr.ndim - 1))
        return pl.BlockSpec(arr.shape, lambda i: (0,) * arr.ndim)

    in_specs = [spec(z, True), spec(x, True)] + [spec(a) for a in args[2:]]

    out = pl.pallas_call(
        _body,
        grid=(B,),
        in_specs=in_specs,
        out_specs=pl.BlockSpec((1, 1, 1), lambda i: (i, 0, 0)),
        out_shape=jax.ShapeDtypeStruct((B, 1, 1), jnp.float32),
    )(*args)
    return out.reshape(B, 1)
